# baseline (speedup 1.0000x reference)
import functools

import jax
import jax.numpy as jnp
import numpy as np
from jax import lax
from jax.experimental import pallas as pl
from jax.experimental.pallas import tpu as pltpu
from jax.experimental.pallas import tpu_sc as plsc

NUM_FIELDS = 26
FIELD_DIM = 100000
TOTAL_ROWS = NUM_FIELDS * FIELD_DIM
EMBED_DIM = 16
BATCH = 16384

NC = 2
NS = 16
NW = NC * NS
BW = BATCH // NW
CB = 64
NCHUNK = BATCH // CB
CPW = BW // CB
IPC = CB * NUM_FIELDS
IG = IPC // 128
GPC = CB // 16
NBUF = 2

_OFFSETS = np.array(
    (0, *np.cumsum([FIELD_DIM] * NUM_FIELDS)[:-1]), dtype=np.int32)


def _fm_body(idx_hbm, embt_hbm, fc_hbm, out_hbm,
             idx_v, rows_v, fc_v, out_v, sem0, sem1):
    wid = lax.axis_index("s") * NC + lax.axis_index("c")
    pltpu.sync_copy(idx_hbm.at[wid], idx_v)
    sems = (sem0, sem1)

    def issue(c, b):
        def g_body(g, _):
            ig = idx_v.at[c * IG + g]
            pltpu.async_copy(
                fc_hbm.at[ig], fc_v.at[b, pl.ds(g * 128, 128)], sems[b])
            for d in range(EMBED_DIM):
                pltpu.async_copy(
                    embt_hbm.at[d].at[ig],
                    rows_v.at[b, d, pl.ds(g * 128, 128)], sems[b])
            return 0

        lax.fori_loop(0, IG, g_body, 0)

    def drain(b):
        pltpu.make_async_copy(
            embt_hbm.at[:, pl.ds(0, IPC)], rows_v.at[b], sems[b]).wait()
        pltpu.make_async_copy(
            fc_hbm.at[pl.ds(0, IPC)], fc_v.at[b], sems[b]).wait()

    def compute(c, b):
        def group_body(g, _):
            o = g * 16
            lin = fc_v[b, pl.ds(o, 16)]
            for f in range(1, NUM_FIELDS):
                lin = lin + fc_v[b, pl.ds(f * CB + o, 16)]

            def d_body(d, carry):
                q, p = carry
                v = rows_v[b, d, pl.ds(o, 16)]
                sd = v
                q = q + v * v
                for f in range(1, NUM_FIELDS):
                    v = rows_v[b, d, pl.ds(f * CB + o, 16)]
                    sd = sd + v
                    q = q + v * v
                return (q, p + sd * sd)

            zero = jnp.zeros((16,), jnp.float32)
            q, p = lax.fori_loop(0, EMBED_DIM, d_body, (zero, zero))
            out_v[pl.ds(c * CB + o, 16)] = lin + 0.5 * (p - q)
            return 0

        lax.fori_loop(0, GPC, group_body, 0)

    for b in range(NBUF):
        issue(b, b)

    def body(i, _):
        c = i * NBUF
        for b in range(NBUF):
            drain(b)
            compute(c + b, b)

            @pl.when(c + b + NBUF < CPW)
            def _():
                issue(c + b + NBUF, b)
        return 0

    lax.fori_loop(0, CPW // NBUF, body, 0)
    pltpu.sync_copy(out_v, out_hbm.at[pl.ds(wid * BW, BW)])


@jax.jit
def _fm(idx_fm, embt, fc_flat):
    mesh = plsc.VectorSubcoreMesh(
        core_axis_name="c", subcore_axis_name="s",
        num_cores=NC, num_subcores=NS)
    f = functools.partial(
        pl.kernel,
        out_type=jax.ShapeDtypeStruct((BATCH,), jnp.float32),
        mesh=mesh,
        compiler_params=pltpu.CompilerParams(
            needs_layout_passes=False, use_tc_tiling_on_sc=False),
        scratch_types=[
            pltpu.VMEM((CPW * IG, 128), jnp.int32),
            pltpu.VMEM((NBUF, EMBED_DIM, IPC), jnp.float32),
            pltpu.VMEM((NBUF, IPC), jnp.float32),
            pltpu.VMEM((BW,), jnp.float32),
            pltpu.SemaphoreType.DMA,
            pltpu.SemaphoreType.DMA,
        ],
    )(_fm_body)
    return f(idx_fm, embt, fc_flat)


def kernel(x, emb_table, fc_table, bias):
    idx = x.astype(jnp.int32) + jnp.asarray(_OFFSETS)[None, :]
    idx_fm = idx.reshape(NCHUNK, CB, NUM_FIELDS).transpose(0, 2, 1)
    idx_fm = idx_fm.reshape(NW, CPW * IG, 128)
    out = _fm(idx_fm, emb_table.T, fc_table[:, 0])
    return out[:, None] + bias[None, :]


# speedup vs baseline: 2.4777x; 2.4777x over previous
import functools

import jax
import jax.numpy as jnp
import numpy as np
from jax import lax
from jax.experimental import pallas as pl
from jax.experimental.pallas import tpu as pltpu
from jax.experimental.pallas import tpu_sc as plsc

NUM_FIELDS = 26
FIELD_DIM = 100000
TOTAL_ROWS = NUM_FIELDS * FIELD_DIM
EMBED_DIM = 16
BATCH = 16384

NC = 2
NS = 16
NW = NC * NS
BW = BATCH // NW
CB = 64
NCHUNK = BATCH // CB
CPW = BW // CB
IPC = CB * NUM_FIELDS
GPC = CB // 16
NBUF = 2
RD = CB * EMBED_DIM

_OFFSETS = np.array(
    (0, *np.cumsum([FIELD_DIM] * NUM_FIELDS)[:-1]), dtype=np.int32)


def _gather_body(idx_hbm, emb_hbm, fc_hbm, gath_hbm, lin_hbm,
                 idx_v, rows_v, fc_v, lin_v, sem0, sem1, semw0, semw1):
    wid = lax.axis_index("s") * NC + lax.axis_index("c")
    pltpu.sync_copy(idx_hbm.at[wid], idx_v)
    sems = (sem0, sem1)
    wsems = (semw0, semw1)

    def issue(c, b):
        ig = idx_v.at[c]
        pltpu.async_copy(emb_hbm.at[ig], rows_v.at[b], sems[b])
        pltpu.async_copy(fc_hbm.at[ig], fc_v.at[b], sems[b])

    def drain(b):
        pltpu.make_async_copy(
            emb_hbm.at[pl.ds(0, IPC)], rows_v.at[b], sems[b]).wait()
        pltpu.make_async_copy(
            fc_hbm.at[pl.ds(0, IPC)], fc_v.at[b], sems[b]).wait()

    for b in range(NBUF):
        issue(b, b)

    def body(i, _):
        c0 = i * NBUF
        for b in range(NBUF):
            c = c0 + b
            drain(b)
            for g in range(GPC):
                o = g * 16
                lin = fc_v[b, pl.ds(o, 16)]
                for f in range(1, NUM_FIELDS):
                    lin = lin + fc_v[b, pl.ds(f * CB + o, 16)]
                lin_v[pl.ds(c * CB + o, 16)] = lin
            pltpu.async_copy(
                rows_v.at[b], gath_hbm.at[wid * CPW + c], wsems[b])

            @pl.when(c + NBUF < CPW)
            def _():
                pltpu.make_async_copy(
                    rows_v.at[b], gath_hbm.at[0], wsems[b]).wait()
                issue(c + NBUF, b)
        return 0

    lax.fori_loop(0, CPW // NBUF, body, 0)
    for b in range(NBUF):
        pltpu.make_async_copy(rows_v.at[b], gath_hbm.at[0], wsems[b]).wait()
    pltpu.sync_copy(lin_v, lin_hbm.at[wid])


def _fm_tc_body(g_ref, lin_ref, out_ref):
    x = g_ref[...]
    s = jnp.sum(x, axis=1)
    q = jnp.sum(x * x, axis=1)
    t = s * s - q
    j = lax.broadcasted_iota(jnp.int32, (RD, CB), 0)
    r = lax.broadcasted_iota(jnp.int32, (RD, CB), 1)
    m = (j // EMBED_DIM == r).astype(jnp.float32)
    fm = jax.lax.dot(t, m, precision=jax.lax.Precision.HIGHEST)
    out_ref[...] = lin_ref[...] + 0.5 * fm


@jax.jit
def _fm(idx_fm, emb, fc_flat):
    mesh = plsc.VectorSubcoreMesh(
        core_axis_name="c", subcore_axis_name="s",
        num_cores=NC, num_subcores=NS)
    f = functools.partial(
        pl.kernel,
        out_type=[
            jax.ShapeDtypeStruct((NCHUNK, IPC, EMBED_DIM), jnp.float32),
            jax.ShapeDtypeStruct((NW, BW), jnp.float32),
        ],
        mesh=mesh,
        compiler_params=pltpu.CompilerParams(
            needs_layout_passes=False, use_tc_tiling_on_sc=False),
        scratch_types=[
            pltpu.VMEM((CPW, IPC), jnp.int32),
            pltpu.VMEM((NBUF, IPC, EMBED_DIM), jnp.float32),
            pltpu.VMEM((NBUF, IPC), jnp.float32),
            pltpu.VMEM((BW,), jnp.float32),
            pltpu.SemaphoreType.DMA,
            pltpu.SemaphoreType.DMA,
            pltpu.SemaphoreType.DMA,
            pltpu.SemaphoreType.DMA,
        ],
    )(_gather_body)
    gath, lin = f(idx_fm, emb, fc_flat)

    CT = 16
    out = pl.pallas_call(
        _fm_tc_body,
        grid=(NCHUNK // CT,),
        in_specs=[
            pl.BlockSpec((CT, NUM_FIELDS, RD), lambda i: (i, 0, 0)),
            pl.BlockSpec((CT, CB), lambda i: (i, 0)),
        ],
        out_specs=pl.BlockSpec((CT, CB), lambda i: (i, 0)),
        out_shape=jax.ShapeDtypeStruct((NCHUNK, CB), jnp.float32),
    )(gath.reshape(NCHUNK, NUM_FIELDS, RD), lin.reshape(NCHUNK, CB))
    return out


def kernel(x, emb_table, fc_table, bias):
    idx = x.astype(jnp.int32) + jnp.asarray(_OFFSETS)[None, :]
    idx_fm = idx.reshape(NCHUNK, CB, NUM_FIELDS).transpose(0, 2, 1)
    idx_fm = idx_fm.reshape(NW, CPW, IPC)
    out = _fm(idx_fm, emb_table, fc_table[:, 0])
    return out.reshape(BATCH)[:, None] + bias[None, :]


# speedup vs baseline: 3.0506x; 1.2312x over previous
import functools

import jax
import jax.numpy as jnp
import numpy as np
from jax import lax
from jax.experimental import pallas as pl
from jax.experimental.pallas import tpu as pltpu
from jax.experimental.pallas import tpu_sc as plsc

NUM_FIELDS = 26
FIELD_DIM = 100000
TOTAL_ROWS = NUM_FIELDS * FIELD_DIM
EMBED_DIM = 16
BATCH = 16384

NC = 2
NS = 16
NW = NC * NS
BW = BATCH // NW
CB = 64
NCHUNK = BATCH // CB
CPW = BW // CB
IPC = CB * NUM_FIELDS
GPC = CB // 16
NBUF = 2

_OFFSETS = np.array(
    (0, *np.cumsum([FIELD_DIM] * NUM_FIELDS)[:-1]), dtype=np.int32)


def _gather_body(idx_hbm, emb_hbm, fc_hbm, t_hbm, lin_hbm,
                 idx_v, rows_v, fc_v, t_v, lin_v, sem0, sem1):
    wid = lax.axis_index("s") * NC + lax.axis_index("c")
    pltpu.sync_copy(idx_hbm.at[wid], idx_v)
    sems = (sem0, sem1)

    def issue(c, b):
        ig = idx_v.at[c]
        pltpu.async_copy(emb_hbm.at[ig], rows_v.at[b], sems[b])
        pltpu.async_copy(fc_hbm.at[ig], fc_v.at[b], sems[b])

    def drain(b):
        pltpu.make_async_copy(
            emb_hbm.at[pl.ds(0, IPC)], rows_v.at[b], sems[b]).wait()
        pltpu.make_async_copy(
            fc_hbm.at[pl.ds(0, IPC)], fc_v.at[b], sems[b]).wait()

    for b in range(NBUF):
        issue(b, b)

    def body(i, _):
        c0 = i * NBUF
        for b in range(NBUF):
            c = c0 + b
            drain(b)
            for g in range(GPC):
                o = g * 16
                lin = fc_v[b, pl.ds(o, 16)]
                for f in range(1, NUM_FIELDS):
                    lin = lin + fc_v[b, pl.ds(f * CB + o, 16)]
                lin_v[pl.ds(c * CB + o, 16)] = lin

            def row_body(r, _):
                s = rows_v[b, r]
                q = s * s
                for f in range(1, NUM_FIELDS):
                    v = rows_v[b, f * CB + r]
                    s = s + v
                    q = q + v * v
                t_v[c * CB + r] = s * s - q
                return 0

            lax.fori_loop(0, CB, row_body, 0)

            @pl.when(c + NBUF < CPW)
            def _():
                issue(c + NBUF, b)
        return 0

    lax.fori_loop(0, CPW // NBUF, body, 0)
    pltpu.sync_copy(t_v, t_hbm.at[wid])
    pltpu.sync_copy(lin_v, lin_hbm.at[wid])


def _fm_tc_body(t_ref, lin_ref, out_ref):
    x = t_ref[...]
    j = lax.broadcasted_iota(jnp.int32, (128, 8), 0)
    k = lax.broadcasted_iota(jnp.int32, (128, 8), 1)
    m = (j // EMBED_DIM == k).astype(jnp.float32)
    fm = jax.lax.dot(x, m, precision=jax.lax.Precision.HIGHEST)
    out_ref[...] = lin_ref[...] + 0.5 * fm


@jax.jit
def _fm(idx_fm, emb, fc_flat):
    mesh = plsc.VectorSubcoreMesh(
        core_axis_name="c", subcore_axis_name="s",
        num_cores=NC, num_subcores=NS)
    f = functools.partial(
        pl.kernel,
        out_type=[
            jax.ShapeDtypeStruct((NW, BW, EMBED_DIM), jnp.float32),
            jax.ShapeDtypeStruct((NW, BW), jnp.float32),
        ],
        mesh=mesh,
        compiler_params=pltpu.CompilerParams(
            needs_layout_passes=False, use_tc_tiling_on_sc=False),
        scratch_types=[
            pltpu.VMEM((CPW, IPC), jnp.int32),
            pltpu.VMEM((NBUF, IPC, EMBED_DIM), jnp.float32),
            pltpu.VMEM((NBUF, IPC), jnp.float32),
            pltpu.VMEM((BW, EMBED_DIM), jnp.float32),
            pltpu.VMEM((BW,), jnp.float32),
            pltpu.SemaphoreType.DMA,
            pltpu.SemaphoreType.DMA,
        ],
    )(_gather_body)
    t, lin = f(idx_fm, emb, fc_flat)

    NR = BATCH * EMBED_DIM // 128
    out = pl.pallas_call(
        _fm_tc_body,
        grid=(1,),
        in_specs=[
            pl.BlockSpec((NR, 128), lambda i: (0, 0)),
            pl.BlockSpec((NR, 8), lambda i: (0, 0)),
        ],
        out_specs=pl.BlockSpec((NR, 8), lambda i: (0, 0)),
        out_shape=jax.ShapeDtypeStruct((NR, 8), jnp.float32),
    )(t.reshape(NR, 128), lin.reshape(NR, 8))
    return out


def kernel(x, emb_table, fc_table, bias):
    idx = x.astype(jnp.int32) + jnp.asarray(_OFFSETS)[None, :]
    idx_fm = idx.reshape(NCHUNK, CB, NUM_FIELDS).transpose(0, 2, 1)
    idx_fm = idx_fm.reshape(NW, CPW, IPC)
    out = _fm(idx_fm, emb_table, fc_table[:, 0])
    return out.reshape(BATCH)[:, None] + bias[None, :]


# speedup vs baseline: 3.0509x; 1.0001x over previous
import functools

import jax
import jax.numpy as jnp
import numpy as np
from jax import lax
from jax.experimental import pallas as pl
from jax.experimental.pallas import tpu as pltpu
from jax.experimental.pallas import tpu_sc as plsc

NUM_FIELDS = 26
FIELD_DIM = 100000
TOTAL_ROWS = NUM_FIELDS * FIELD_DIM
EMBED_DIM = 16
BATCH = 16384

NC = 2
NS = 16
NW = NC * NS
BW = BATCH // NW
CB = 64
NCHUNK = BATCH // CB
CPW = BW // CB
IPC = CB * NUM_FIELDS
GPC = CB // 16
NBUF = 2

_OFFSETS = np.array(
    (0, *np.cumsum([FIELD_DIM] * NUM_FIELDS)[:-1]), dtype=np.int32)


def _gather_body(idx_hbm, emb_hbm, fc_hbm, t_hbm, lin_hbm,
                 idx_v, rows_v, fc_v, t_v, lin_v, sem0, sem1):
    wid = lax.axis_index("s") * NC + lax.axis_index("c")
    pltpu.sync_copy(idx_hbm.at[wid], idx_v)
    sems = (sem0, sem1)

    def issue(c, b):
        ig = idx_v.at[c]
        pltpu.async_copy(emb_hbm.at[ig], rows_v.at[b], sems[b])
        pltpu.async_copy(fc_hbm.at[ig], fc_v.at[b], sems[b])

    def drain(b):
        pltpu.make_async_copy(
            emb_hbm.at[pl.ds(0, IPC)], rows_v.at[b], sems[b]).wait()
        pltpu.make_async_copy(
            fc_hbm.at[pl.ds(0, IPC)], fc_v.at[b], sems[b]).wait()

    for b in range(NBUF):
        issue(b, b)

    def body(i, _):
        c0 = i * NBUF
        for b in range(NBUF):
            c = c0 + b
            drain(b)
            ir = lax.iota(jnp.int32, 16)
            for g in range(GPC):
                o = g * 16
                base16 = (o + ir) * NUM_FIELDS
                lin = plsc.load_gather(fc_v.at[b], [base16])
                for f in range(1, NUM_FIELDS):
                    lin = lin + plsc.load_gather(fc_v.at[b], [base16 + f])
                lin_v[pl.ds(c * CB + o, 16)] = lin

            @plsc.parallel_loop(0, CB, unroll=4)
            def row_body(r):
                base = r * NUM_FIELDS
                s = rows_v[b, base]
                q = s * s
                for f in range(1, NUM_FIELDS):
                    v = rows_v[b, base + f]
                    s = s + v
                    q = q + v * v
                t_v[c * CB + r] = s * s - q

            @pl.when(c + NBUF < CPW)
            def _():
                issue(c + NBUF, b)
        return 0

    lax.fori_loop(0, CPW // NBUF, body, 0)
    pltpu.sync_copy(t_v, t_hbm.at[wid])
    pltpu.sync_copy(lin_v, lin_hbm.at[wid])


def _fm_tc_body(t_ref, lin_ref, out_ref):
    x = t_ref[...]
    j = lax.broadcasted_iota(jnp.int32, (128, 8), 0)
    k = lax.broadcasted_iota(jnp.int32, (128, 8), 1)
    m = (j // EMBED_DIM == k).astype(jnp.float32)
    fm = jax.lax.dot(x, m, precision=jax.lax.Precision.HIGHEST)
    out_ref[...] = lin_ref[...] + 0.5 * fm


@jax.jit
def _fm(idx_fm, emb, fc_flat):
    mesh = plsc.VectorSubcoreMesh(
        core_axis_name="c", subcore_axis_name="s",
        num_cores=NC, num_subcores=NS)
    f = functools.partial(
        pl.kernel,
        out_type=[
            jax.ShapeDtypeStruct((NW, BW, EMBED_DIM), jnp.float32),
            jax.ShapeDtypeStruct((NW, BW), jnp.float32),
        ],
        mesh=mesh,
        compiler_params=pltpu.CompilerParams(
            needs_layout_passes=False, use_tc_tiling_on_sc=False),
        scratch_types=[
            pltpu.VMEM((CPW, IPC), jnp.int32),
            pltpu.VMEM((NBUF, IPC, EMBED_DIM), jnp.float32),
            pltpu.VMEM((NBUF, IPC), jnp.float32),
            pltpu.VMEM((BW, EMBED_DIM), jnp.float32),
            pltpu.VMEM((BW,), jnp.float32),
            pltpu.SemaphoreType.DMA,
            pltpu.SemaphoreType.DMA,
        ],
    )(_gather_body)
    t, lin = f(idx_fm, emb, fc_flat)

    NR = BATCH * EMBED_DIM // 128
    out = pl.pallas_call(
        _fm_tc_body,
        grid=(1,),
        in_specs=[
            pl.BlockSpec((NR, 128), lambda i: (0, 0)),
            pl.BlockSpec((NR, 8), lambda i: (0, 0)),
        ],
        out_specs=pl.BlockSpec((NR, 8), lambda i: (0, 0)),
        out_shape=jax.ShapeDtypeStruct((NR, 8), jnp.float32),
    )(t.reshape(NR, 128), lin.reshape(NR, 8))
    return out


def kernel(x, emb_table, fc_table, bias):
    idx = x.astype(jnp.int32) + jnp.asarray(_OFFSETS)[None, :]
    idx_fm = idx.reshape(NW, CPW, IPC)
    out = _fm(idx_fm, emb_table, fc_table[:, 0])
    return out.reshape(BATCH)[:, None] + bias[None, :]


# speedup vs baseline: 3.0572x; 1.0021x over previous
import functools

import jax
import jax.numpy as jnp
import numpy as np
from jax import lax
from jax.experimental import pallas as pl
from jax.experimental.pallas import tpu as pltpu
from jax.experimental.pallas import tpu_sc as plsc

NUM_FIELDS = 26
FIELD_DIM = 100000
TOTAL_ROWS = NUM_FIELDS * FIELD_DIM
EMBED_DIM = 16
BATCH = 16384

NC = 2
NS = 16
NW = NC * NS
BW = BATCH // NW
CB = 32
NCHUNK = BATCH // CB
CPW = BW // CB
IPC = CB * NUM_FIELDS
GPC = CB // 16
NBUF = 4

_OFFSETS = np.array(
    (0, *np.cumsum([FIELD_DIM] * NUM_FIELDS)[:-1]), dtype=np.int32)


def _gather_body(idx_hbm, emb_hbm, fc_hbm, t_hbm, lin_hbm,
                 idx_v, rows_v, fc_v, t_v, lin_v, *sems):
    wid = lax.axis_index("s") * NC + lax.axis_index("c")
    pltpu.sync_copy(idx_hbm.at[wid], idx_v)

    def issue(c, b):
        ig = idx_v.at[c]
        pltpu.async_copy(emb_hbm.at[ig], rows_v.at[b], sems[b])
        pltpu.async_copy(fc_hbm.at[ig], fc_v.at[b], sems[b])

    def drain(b):
        pltpu.make_async_copy(
            emb_hbm.at[pl.ds(0, IPC)], rows_v.at[b], sems[b]).wait()
        pltpu.make_async_copy(
            fc_hbm.at[pl.ds(0, IPC)], fc_v.at[b], sems[b]).wait()

    for b in range(NBUF):
        issue(b, b)

    def body(i, _):
        c0 = i * NBUF
        for b in range(NBUF):
            c = c0 + b
            drain(b)
            ir = lax.iota(jnp.int32, 16)
            for g in range(GPC):
                o = g * 16
                base16 = (o + ir) * NUM_FIELDS
                lin = plsc.load_gather(fc_v.at[b], [base16])
                for f in range(1, NUM_FIELDS):
                    lin = lin + plsc.load_gather(fc_v.at[b], [base16 + f])
                lin_v[pl.ds(c * CB + o, 16)] = lin

            @plsc.parallel_loop(0, CB, unroll=4)
            def row_body(r):
                base = r * NUM_FIELDS
                s = rows_v[b, base]
                q = s * s
                for f in range(1, NUM_FIELDS):
                    v = rows_v[b, base + f]
                    s = s + v
                    q = q + v * v
                t_v[c * CB + r] = s * s - q

            @pl.when(c + NBUF < CPW)
            def _():
                issue(c + NBUF, b)
        return 0

    lax.fori_loop(0, CPW // NBUF, body, 0)
    pltpu.sync_copy(t_v, t_hbm.at[wid])
    pltpu.sync_copy(lin_v, lin_hbm.at[wid])


def _fm_tc_body(t_ref, lin_ref, out_ref):
    x = t_ref[...]
    j = lax.broadcasted_iota(jnp.int32, (128, 8), 0)
    k = lax.broadcasted_iota(jnp.int32, (128, 8), 1)
    m = (j // EMBED_DIM == k).astype(jnp.float32)
    fm = jax.lax.dot(x, m, precision=jax.lax.Precision.HIGHEST)
    out_ref[...] = lin_ref[...] + 0.5 * fm


@jax.jit
def _fm(idx_fm, emb, fc_flat):
    mesh = plsc.VectorSubcoreMesh(
        core_axis_name="c", subcore_axis_name="s",
        num_cores=NC, num_subcores=NS)
    f = functools.partial(
        pl.kernel,
        out_type=[
            jax.ShapeDtypeStruct((NW, BW, EMBED_DIM), jnp.float32),
            jax.ShapeDtypeStruct((NW, BW), jnp.float32),
        ],
        mesh=mesh,
        compiler_params=pltpu.CompilerParams(
            needs_layout_passes=False, use_tc_tiling_on_sc=False),
        scratch_types=[
            pltpu.VMEM((CPW, IPC), jnp.int32),
            pltpu.VMEM((NBUF, IPC, EMBED_DIM), jnp.float32),
            pltpu.VMEM((NBUF, IPC), jnp.float32),
            pltpu.VMEM((BW, EMBED_DIM), jnp.float32),
            pltpu.VMEM((BW,), jnp.float32),
        ] + [pltpu.SemaphoreType.DMA] * NBUF,
    )(_gather_body)
    t, lin = f(idx_fm, emb, fc_flat)

    NR = BATCH * EMBED_DIM // 128
    out = pl.pallas_call(
        _fm_tc_body,
        grid=(1,),
        in_specs=[
            pl.BlockSpec((NR, 128), lambda i: (0, 0)),
            pl.BlockSpec((NR, 8), lambda i: (0, 0)),
        ],
        out_specs=pl.BlockSpec((NR, 8), lambda i: (0, 0)),
        out_shape=jax.ShapeDtypeStruct((NR, 8), jnp.float32),
    )(t.reshape(NR, 128), lin.reshape(NR, 8))
    return out


def kernel(x, emb_table, fc_table, bias):
    idx = x.astype(jnp.int32) + jnp.asarray(_OFFSETS)[None, :]
    idx_fm = idx.reshape(NW, CPW, IPC)
    out = _fm(idx_fm, emb_table, fc_table[:, 0])
    return out.reshape(BATCH)[:, None] + bias[None, :]
